# baseline (speedup 1.0000x reference)
import functools

import jax
import jax.numpy as jnp
from jax import lax
from jax.experimental import pallas as pl
from jax.experimental.pallas import tpu as pltpu
from jax.experimental.pallas import tpu_sc as plsc

N_NODES = 50000
E = 800000
ND = 16
NUM_LAYER = 8

NC, NS = 2, 16
NW = NC * NS
EPW = E // NW
CH_A = 5000
NPW = 1563
NPAD = NW * NPW
CH_C = 4000
CHP = CH_C + 1

@functools.cache
def _mesh():
    return plsc.VectorSubcoreMesh(core_axis_name="c", subcore_axis_name="s",
                                  num_cores=NC, num_subcores=NS)

BE = 6400
GRID = E // BE


def _wid():
    return lax.axis_index("s") * NC + lax.axis_index("c")


def _gather2_body(tab, idx1, idx2, out1, out2, idxv, buf, sem):
    w = _wid()

    def chunk(k, carry):
        off = w * EPW + k * CH_A
        for idx_hbm, out_hbm in ((idx1, out1), (idx2, out2)):
            pltpu.sync_copy(idx_hbm.at[pl.ds(off, CH_A)], idxv)
            pltpu.async_copy(tab.at[idxv], buf, sem).wait()
            cps = [pltpu.async_copy(buf.at[:, f],
                                    out_hbm.at[f, pl.ds(off, CH_A)], sem)
                   for f in range(ND)]
            for cp in cps:
                cp.wait()
        return carry

    lax.fori_loop(0, EPW // CH_A, chunk, 0)


def _gather1_body(tab, idx1, out1, idxv, buf, sem):
    w = _wid()

    def chunk(k, carry):
        off = w * EPW + k * CH_A
        pltpu.sync_copy(idx1.at[pl.ds(off, CH_A)], idxv)
        pltpu.async_copy(tab.at[idxv], buf, sem).wait()
        cps = [pltpu.async_copy(buf.at[:, f],
                                out1.at[f, pl.ds(off, CH_A)], sem)
               for f in range(ND)]
        for cp in cps:
            cp.wait()
        return carry

    lax.fori_loop(0, EPW // CH_A, chunk, 0)


def _gather2(tab, idx1, idx2):
    return pl.kernel(
        _gather2_body,
        mesh=_mesh(),
        out_type=[jax.ShapeDtypeStruct((ND, E), jnp.float32),
                  jax.ShapeDtypeStruct((ND, E), jnp.float32)],
        scratch_types=[pltpu.VMEM((CH_A,), jnp.int32),
                       pltpu.VMEM((CH_A, ND), jnp.float32),
                       pltpu.SemaphoreType.DMA],
    )(tab, idx1, idx2)


def _gather1(tab, idx1):
    return pl.kernel(
        _gather1_body,
        mesh=_mesh(),
        out_type=jax.ShapeDtypeStruct((ND, E), jnp.float32),
        scratch_types=[pltpu.VMEM((CH_A,), jnp.int32),
                       pltpu.VMEM((CH_A, ND), jnp.float32),
                       pltpu.SemaphoreType.DMA],
    )(tab, idx1)


def _segmax_body(mT, dst_s, bounds, mout, boundsv, dstv, mbuf, table, sem):
    w = _wid()
    pltpu.sync_copy(bounds, boundsv)

    def zrow(i, carry):
        table[i, :] = jnp.zeros((ND,), jnp.float32)
        return carry

    lax.fori_loop(0, NPW, zrow, 0)

    bv = boundsv[pl.ds(w, 16)]
    lo = bv[0]
    hi = bv[1]
    nbase = w * NPW
    c0 = lo // CH_C
    c1 = (hi + CH_C - 1) // CH_C
    lanes = lax.iota(jnp.int32, 16)

    def chunk(c, carry):
        base = c * CH_C
        pltpu.sync_copy(dst_s.at[pl.ds(base, CH_C)], dstv.at[pl.ds(0, CH_C)])
        pltpu.sync_copy(mT.at[:, pl.ds(base, CH_C)], mbuf.at[:, pl.ds(0, CH_C)])
        s = jnp.maximum(lo - base, 0)
        t = jnp.minimum(hi - base, CH_C)

        def edge(e, car):
            cur_d, acc = car
            d = dstv[pl.ds(e, 16)][0]
            row = plsc.load_gather(mbuf, [lanes, jnp.full((16,), e, jnp.int32)])
            is_new = d != cur_d
            acc = jnp.where(is_new, row, jnp.maximum(acc, row))
            table[d - nbase, :] = acc
            return (d, acc)

        return lax.fori_loop(s, t, edge, carry)

    lax.fori_loop(c0, c1, chunk,
                  (jnp.int32(-1), jnp.full((16,), -jnp.inf, jnp.float32)))
    pltpu.sync_copy(table, mout.at[pl.ds(nbase, NPW)])


def _segmax(mT, dst_s, bounds):
    return pl.kernel(
        _segmax_body,
        mesh=_mesh(),
        out_type=jax.ShapeDtypeStruct((NPAD, ND), jnp.float32),
        scratch_types=[pltpu.VMEM((48,), jnp.int32),
                       pltpu.VMEM((CH_C + 16,), jnp.int32),
                       pltpu.VMEM((ND, CHP), jnp.float32),
                       pltpu.VMEM((NPW, ND), jnp.float32),
                       pltpu.SemaphoreType.DMA],
    )(mT, dst_s, bounds)


def _scatterp_body(osrt, perm, out, idxv, valv, sem):
    w = _wid()

    def chunk(k, carry):
        off = w * EPW + k * CH_A
        pltpu.sync_copy(perm.at[pl.ds(off, CH_A)], idxv)
        pltpu.sync_copy(osrt.at[pl.ds(off, CH_A)], valv)
        pltpu.async_copy(valv, out.at[idxv], sem).wait()
        return carry

    lax.fori_loop(0, EPW // CH_A, chunk, 0)


def _scatterp(osrt, perm):
    return pl.kernel(
        _scatterp_body,
        mesh=_mesh(),
        out_type=jax.ShapeDtypeStruct((E,), jnp.float32),
        scratch_types=[pltpu.VMEM((CH_A,), jnp.int32),
                       pltpu.VMEM((CH_A,), jnp.float32),
                       pltpu.SemaphoreType.DMA],
    )(osrt, perm)


def _mlp_body(we1, be1, we2, be2, wv1, bv1, wv2, bv2, miT, mjT, hT, hout, mout):
    x = jnp.concatenate([miT[...], mjT[...], hT[...]], axis=0)
    t = jax.lax.dot_general(we1[...], x, (((1,), (0,)), ((), ())),
                            preferred_element_type=jnp.float32)
    t = jnp.maximum(t + be1[...], 0.0)
    h = jax.lax.dot_general(we2[...], t, (((1,), (0,)), ((), ())),
                            preferred_element_type=jnp.float32) + be2[...]
    y = jnp.concatenate([miT[...], h], axis=0)
    u = jax.lax.dot_general(wv1[...], y, (((1,), (0,)), ((), ())),
                            preferred_element_type=jnp.float32)
    u = jnp.maximum(u + bv1[...], 0.0)
    m = jax.lax.dot_general(wv2[...], u, (((1,), (0,)), ((), ())),
                            preferred_element_type=jnp.float32) + bv2[...]
    hout[...] = h
    mout[...] = m


def _full(shape):
    return pl.BlockSpec(shape, lambda i: (0,) * len(shape))


_EDGE_SPEC = pl.BlockSpec((ND, BE), lambda i: (0, i))


def _mlp_layer(we1, be1, we2, be2, wv1, bv1, wv2, bv2, miT, mjT, hT):
    return pl.pallas_call(
        _mlp_body,
        grid=(GRID,),
        in_specs=[
            _full((24, 48)), _full((24, 1)), _full((16, 24)), _full((16, 1)),
            _full((16, 32)), _full((16, 1)), _full((16, 16)), _full((16, 1)),
            _EDGE_SPEC, _EDGE_SPEC, _EDGE_SPEC,
        ],
        out_specs=[_EDGE_SPEC, _EDGE_SPEC],
        out_shape=[
            jax.ShapeDtypeStruct((ND, E), jnp.float32),
            jax.ShapeDtypeStruct((ND, E), jnp.float32),
        ],
    )(we1, be1, we2, be2, wv1, bv1, wv2, bv2, miT, mjT, hT)


def _proj_body(wf, bf, hT, oout):
    o = jax.lax.dot_general(wf[...], hT[...], (((1,), (0,)), ((), ())),
                            preferred_element_type=jnp.float32)
    oout[...] = jnp.maximum(o + bf[...], 0.0)


def _final_proj(wf, bf, hT):
    return pl.pallas_call(
        _proj_body,
        grid=(GRID,),
        in_specs=[_full((1, 16)), _full((1, 1)), _EDGE_SPEC],
        out_specs=pl.BlockSpec((1, BE), lambda i: (0, i)),
        out_shape=jax.ShapeDtypeStruct((1, E), jnp.float32),
    )(wf, bf, hT)


def kernel(M, H, edge_index, We1, be1, We2, be2, Wv1, bv1, Wv2, bv2, Wf, bf):
    src = edge_index[0]
    dst = edge_index[1]
    perm = jnp.argsort(dst).astype(jnp.int32)
    dst_s = jnp.take(dst, perm)
    src_s = jnp.take(src, perm)
    starts = (jnp.arange(NW + 1, dtype=jnp.int32) * NPW).astype(jnp.int32)
    bounds = jnp.searchsorted(dst_s, starts).astype(jnp.int32)
    bounds = jnp.concatenate(
        [bounds, jnp.full((48 - NW - 1,), E, jnp.int32)])

    be1c = be1.reshape(24, 1)
    be2c = be2.reshape(16, 1)
    bv1c = bv1.reshape(16, 1)
    bv2c = bv2.reshape(16, 1)
    bfc = bf.reshape(1, 1)

    HT = jnp.take(H, perm, axis=0).T
    Mcur = M
    for l in range(NUM_LAYER):
        MiT = jnp.take(Mcur, dst_s, axis=0).T
        MjT = jnp.take(Mcur, src_s, axis=0).T
        hT, mT = _mlp_layer(We1, be1c, We2, be2c, Wv1, bv1c, Wv2, bv2c,
                            MiT, MjT, HT)
        HT = hT
        if l < NUM_LAYER - 1:
            Magg = jax.ops.segment_max(mT.T, dst_s, num_segments=N_NODES)
            Mcur = jnp.where(jnp.isneginf(Magg), 0.0, Magg)

    osrt = _final_proj(Wf, bfc, HT)
    inv = jnp.argsort(perm)
    out = jnp.take(osrt.reshape(E), inv)
    return out.reshape(E, 1)


# speedup vs baseline: 1.0777x; 1.0777x over previous
import functools

import jax
import jax.numpy as jnp
from jax import lax
from jax.experimental import pallas as pl
from jax.experimental.pallas import tpu as pltpu
from jax.experimental.pallas import tpu_sc as plsc

N_NODES = 50000
E = 800000
ND = 16
NUM_LAYER = 8

NC, NS = 2, 16
NW = NC * NS
CH_A = 2000
NCH_A = E // CH_A
KMAX_A = (NCH_A + NW - 1) // NW
NPW = 1563
NPAD = NW * NPW
CH_C = 4000
CHP = CH_C + 1

_SC_PARAMS = pltpu.CompilerParams(use_tc_tiling_on_sc=False,
                                  needs_layout_passes=False)

BE = 6400
GRID = E // BE


@functools.cache
def _mesh():
    return plsc.VectorSubcoreMesh(core_axis_name="c", subcore_axis_name="s",
                                  num_cores=NC, num_subcores=NS)


def _wid():
    return lax.axis_index("s") * NC + lax.axis_index("c")


def _gather_body(n_idx, tab, *args):
    idxs = args[:n_idx]
    outs = args[n_idx:2 * n_idx]
    idxv, buf, tbuf, sem = args[2 * n_idx:]
    w = _wid()
    lanes = lax.iota(jnp.int32, 16)

    def chunk(k, carry):
        c = w + k * NW

        @pl.when(c < NCH_A)
        def _():
            off = c * CH_A
            for idx_hbm, out_hbm in zip(idxs, outs):
                pltpu.sync_copy(idx_hbm.at[pl.ds(off, CH_A)], idxv)
                pltpu.async_copy(tab.at[idxv], buf, sem).wait()

                def tp(j, c2):
                    row = buf[j, :]
                    plsc.store_scatter(
                        tbuf, [lanes, jnp.full((16,), j, jnp.int32)], row)
                    return c2

                lax.fori_loop(0, CH_A, tp, 0)
                pltpu.sync_copy(tbuf.at[:, pl.ds(0, CH_A)],
                                out_hbm.at[:, pl.ds(off, CH_A)])

        return carry

    lax.fori_loop(0, KMAX_A, chunk, 0)


def _gather(tab, *idxs):
    n = len(idxs)
    out = [jax.ShapeDtypeStruct((ND, E), jnp.float32)] * n
    res = pl.kernel(
        functools.partial(_gather_body, n),
        mesh=_mesh(),
        out_type=out if n > 1 else out[0],
        compiler_params=_SC_PARAMS,
        scratch_types=[pltpu.VMEM((CH_A,), jnp.int32),
                       pltpu.VMEM((CH_A, ND), jnp.float32),
                       pltpu.VMEM((ND, CH_A + 1), jnp.float32),
                       pltpu.SemaphoreType.DMA],
    )(tab, *idxs)
    return res


def _segmax_body(mT, dst_s, bounds, mout, boundsv, dstv, mbuf, table, sem):
    w = _wid()
    pltpu.sync_copy(bounds, boundsv)
    zero = jnp.zeros((16,), jnp.float32)

    def zrow(i, carry):
        table[pl.ds(i * 16, 16)] = zero
        return carry

    lax.fori_loop(0, NPW, zrow, 0)

    bv = boundsv[pl.ds(w, 16)]
    lo = bv[0]
    hi = bv[1]
    nbase = w * NPW
    lanes = lax.iota(jnp.int32, 16)

    def chunk(c, carry):
        base = c * CH_C
        pltpu.sync_copy(dst_s.at[pl.ds(base, CH_C)], dstv.at[pl.ds(0, CH_C)])
        pltpu.sync_copy(mT.at[:, pl.ds(base, CH_C)], mbuf.at[:, pl.ds(0, CH_C)])
        s = jnp.maximum(lo - base, 0)
        t = jnp.minimum(hi - base, CH_C)

        def edge(e, car):
            cur_d, acc = car
            d = dstv[pl.ds(e, 16)][0]
            row = plsc.load_gather(mbuf, [lanes, jnp.full((16,), e, jnp.int32)])
            is_new = d != cur_d
            acc = jnp.where(is_new, row, jnp.maximum(acc, row))
            table[pl.ds((d - nbase) * 16, 16)] = acc
            return (d, acc)

        return lax.fori_loop(s, t, edge, carry)

    lax.fori_loop(lo // CH_C, (hi + CH_C - 1) // CH_C, chunk,
                  (jnp.int32(-1), jnp.full((16,), -jnp.inf, jnp.float32)))
    pltpu.sync_copy(table, mout.at[pl.ds(nbase * 16, NPW * 16)])


def _segmax(mT, dst_s, bounds):
    return pl.kernel(
        _segmax_body,
        mesh=_mesh(),
        out_type=jax.ShapeDtypeStruct((NPAD * 16,), jnp.float32),
        compiler_params=_SC_PARAMS,
        scratch_types=[pltpu.VMEM((48,), jnp.int32),
                       pltpu.VMEM((CH_C + 16,), jnp.int32),
                       pltpu.VMEM((ND, CHP), jnp.float32),
                       pltpu.VMEM((NPW * 16,), jnp.float32),
                       pltpu.SemaphoreType.DMA],
    )(mT, dst_s, bounds)


def _scatterp_body(osrt, perm, out, idxv, valv, sem):
    w = _wid()

    def chunk(k, carry):
        c = w + k * NW

        @pl.when(c < NCH_A)
        def _():
            off = c * CH_A
            pltpu.sync_copy(perm.at[pl.ds(off, CH_A)], idxv)
            pltpu.sync_copy(osrt.at[pl.ds(off, CH_A)], valv)
            pltpu.async_copy(valv, out.at[idxv], sem).wait()

        return carry

    lax.fori_loop(0, KMAX_A, chunk, 0)


def _scatterp(osrt, perm):
    return pl.kernel(
        _scatterp_body,
        mesh=_mesh(),
        out_type=jax.ShapeDtypeStruct((E,), jnp.float32),
        compiler_params=_SC_PARAMS,
        scratch_types=[pltpu.VMEM((CH_A,), jnp.int32),
                       pltpu.VMEM((CH_A,), jnp.float32),
                       pltpu.SemaphoreType.DMA],
    )(osrt, perm)


def _mlp_body(we1, be1, we2, be2, wv1, bv1, wv2, bv2, miT, mjT, hT, hout, mout):
    x = jnp.concatenate([miT[...], mjT[...], hT[...]], axis=0)
    t = jax.lax.dot_general(we1[...], x, (((1,), (0,)), ((), ())),
                            preferred_element_type=jnp.float32)
    t = jnp.maximum(t + be1[...], 0.0)
    h = jax.lax.dot_general(we2[...], t, (((1,), (0,)), ((), ())),
                            preferred_element_type=jnp.float32) + be2[...]
    y = jnp.concatenate([miT[...], h], axis=0)
    u = jax.lax.dot_general(wv1[...], y, (((1,), (0,)), ((), ())),
                            preferred_element_type=jnp.float32)
    u = jnp.maximum(u + bv1[...], 0.0)
    m = jax.lax.dot_general(wv2[...], u, (((1,), (0,)), ((), ())),
                            preferred_element_type=jnp.float32) + bv2[...]
    hout[...] = h
    mout[...] = m


def _full(shape):
    return pl.BlockSpec(shape, lambda i: (0,) * len(shape))


_EDGE_SPEC = pl.BlockSpec((ND, BE), lambda i: (0, i))


def _mlp_layer(we1, be1, we2, be2, wv1, bv1, wv2, bv2, miT, mjT, hT):
    return pl.pallas_call(
        _mlp_body,
        grid=(GRID,),
        in_specs=[
            _full((24, 48)), _full((24, 1)), _full((16, 24)), _full((16, 1)),
            _full((16, 32)), _full((16, 1)), _full((16, 16)), _full((16, 1)),
            _EDGE_SPEC, _EDGE_SPEC, _EDGE_SPEC,
        ],
        out_specs=[_EDGE_SPEC, _EDGE_SPEC],
        out_shape=[
            jax.ShapeDtypeStruct((ND, E), jnp.float32),
            jax.ShapeDtypeStruct((ND, E), jnp.float32),
        ],
    )(we1, be1, we2, be2, wv1, bv1, wv2, bv2, miT, mjT, hT)


def _proj_body(wf, bf, hT, oout):
    o = jax.lax.dot_general(wf[...], hT[...], (((1,), (0,)), ((), ())),
                            preferred_element_type=jnp.float32)
    oout[...] = jnp.maximum(o + bf[...], 0.0)


def _final_proj(wf, bf, hT):
    return pl.pallas_call(
        _proj_body,
        grid=(GRID,),
        in_specs=[_full((1, 16)), _full((1, 1)), _EDGE_SPEC],
        out_specs=pl.BlockSpec((1, BE), lambda i: (0, i)),
        out_shape=jax.ShapeDtypeStruct((1, E), jnp.float32),
    )(wf, bf, hT)


def kernel(M, H, edge_index, We1, be1, We2, be2, Wv1, bv1, Wv2, bv2, Wf, bf):
    src = edge_index[0]
    dst = edge_index[1]
    perm = jnp.argsort(dst).astype(jnp.int32)
    dst_s = jnp.take(dst, perm)
    src_s = jnp.take(src, perm)
    starts = (jnp.arange(NW + 1, dtype=jnp.int32) * NPW).astype(jnp.int32)
    bounds = jnp.searchsorted(dst_s, starts).astype(jnp.int32)
    bounds = jnp.concatenate(
        [bounds, jnp.full((48 - NW - 1,), E, jnp.int32)])

    be1c = be1.reshape(24, 1)
    be2c = be2.reshape(16, 1)
    bv1c = bv1.reshape(16, 1)
    bv2c = bv2.reshape(16, 1)
    bfc = bf.reshape(1, 1)

    HT = _gather(H, perm)
    Mcur = jnp.pad(M, ((0, NPAD - N_NODES), (0, 0)))
    for l in range(NUM_LAYER):
        MiT, MjT = _gather(Mcur, dst_s, src_s)
        hT, mT = _mlp_layer(We1, be1c, We2, be2c, Wv1, bv1c, Wv2, bv2c,
                            MiT, MjT, HT)
        HT = hT
        if l < NUM_LAYER - 1:
            Mcur = _segmax(mT, dst_s, bounds).reshape(NPAD, ND)

    osrt = _final_proj(Wf, bfc, HT)
    out = _scatterp(osrt.reshape(E), perm)
    return out.reshape(E, 1)
